# baseline (speedup 1.0000x reference)
import jax
import jax.numpy as jnp
from jax import lax
from jax.experimental import pallas as pl
from jax.experimental.pallas import tpu as pltpu
from jax.experimental.pallas import tpu_sc as plsc

_POINT = 11
_PAD_P = 16
_ACT = 4
_GROUPS = ((0, 4), (4, 9), (9, 10), (10, 11))

_BLK = 8192

_SC_INFO = plsc.get_sparse_core_info()
_NC = _SC_INFO.num_cores
_NW = _NC * _SC_INFO.num_subcores
_LANES = _SC_INFO.num_lanes


def _tc_body(x_ref, av_ref, w_ref, b_ref, out_ref, wb_ref):
    x = x_ref[...]
    av = av_ref[0]
    avn2 = jnp.sum(av * av, axis=0, keepdims=True)
    avs = av / jnp.maximum(jnp.sqrt(avn2), 1e-8)
    numT = lax.dot_general(avs, x, (((0,), (1,)), ((), ())),
                           preferred_element_type=jnp.float32)
    ones = jnp.ones((1, x_ref.shape[1]), jnp.float32)
    ssT = lax.dot_general(ones, x * x, (((1,), (1,)), ((), ())),
                          preferred_element_type=jnp.float32)
    simsT = numT / jnp.maximum(jnp.sqrt(ssT), 1e-8)
    out_ref[...] = jnp.concatenate(
        [simsT, jnp.zeros((_PAD_P - _POINT, x_ref.shape[0]), jnp.float32)], axis=0)

    @pl.when(pl.program_id(0) == 0)
    def _():
        w = w_ref[...]
        lane = lax.broadcasted_iota(jnp.int32, (1, _LANES), 1)
        row_b = jnp.zeros((1, _LANES), jnp.float32)
        for j in range(_ACT):
            row_b = jnp.where(lane == j, b_ref[j], row_b)
        pad_c = jnp.zeros((_ACT, _LANES - _ACT), jnp.float32)
        wb_ref[...] = jnp.concatenate(
            [jnp.concatenate([w, pad_c], axis=1),
             row_b,
             jnp.zeros((3, _LANES), jnp.float32)], axis=0)


def _sc_body(sims_hbm, wb_hbm, out_hbm, sims_v, wb_v, out_v, sem0, sem1):
    cols = sims_v.shape[1]
    half = cols // 2
    wid = lax.axis_index("s") * _NC + lax.axis_index("c")
    base = wid * cols
    cp0 = pltpu.async_copy(sims_hbm.at[:, pl.ds(base, half)],
                           sims_v.at[:, pl.ds(0, half)], sem0)
    cp1 = pltpu.async_copy(sims_hbm.at[:, pl.ds(base + half, half)],
                           sims_v.at[:, pl.ds(half, half)], sem1)
    pltpu.sync_copy(wb_hbm, wb_v)
    wrows = [wb_v[j] for j in range(_ACT + 1)]
    w = [[wrows[j][k] for k in range(_ACT)] for j in range(_ACT)]
    bias = [wrows[_ACT][j] for j in range(_ACT)]
    def _chunk(i):
        sl = pl.ds(i, _LANES)
        v = [sims_v[r, sl] for r in range(_POINT)]
        g = []
        for (s, e) in _GROUPS:
            m = v[s]
            for r in range(s + 1, e):
                m = jnp.maximum(m, v[r])
            g.append(m)
        logits = []
        for j in range(_ACT):
            l = bias[j]
            for k in range(_ACT):
                l = l + w[j][k] * g[k]
            logits.append(l)
        m = jnp.maximum(jnp.maximum(logits[0], logits[1]),
                        jnp.maximum(logits[2], logits[3]))
        exps = [jnp.exp(l - m) for l in logits]
        inv = 1.0 / (exps[0] + exps[1] + exps[2] + exps[3])
        for j in range(_ACT):
            out_v[j, sl] = exps[j] * inv

    cp0.wait()
    plsc.parallel_loop(0, half, _LANES, unroll=4)(_chunk)
    cp1.wait()
    plsc.parallel_loop(half, cols, _LANES, unroll=4)(_chunk)
    pltpu.sync_copy(out_v, out_hbm.at[:, pl.ds(base, cols)])


def kernel(embedded_words, action_vectors, W, b):
    B, D = embedded_words.shape
    cols = B // _NW

    simsT, wb = pl.pallas_call(
        _tc_body,
        grid=(B // _BLK,),
        in_specs=[
            pl.BlockSpec((_BLK, D), lambda i: (i, 0)),
            pl.BlockSpec((1, D, _POINT), lambda i: (0, 0, 0)),
            pl.BlockSpec((_ACT, _ACT), lambda i: (0, 0)),
            pl.BlockSpec(memory_space=pltpu.SMEM),
        ],
        out_specs=[
            pl.BlockSpec((_PAD_P, _BLK), lambda i: (0, i)),
            pl.BlockSpec((8, _LANES), lambda i: (0, 0)),
        ],
        out_shape=[
            jax.ShapeDtypeStruct((_PAD_P, B), jnp.float32),
            jax.ShapeDtypeStruct((8, _LANES), jnp.float32),
        ],
    )(embedded_words, action_vectors, W, b)

    sc_fn = pl.kernel(
        _sc_body,
        out_type=jax.ShapeDtypeStruct((_ACT, B), jnp.float32),
        mesh=plsc.VectorSubcoreMesh(core_axis_name="c", subcore_axis_name="s"),
        scratch_types=[
            pltpu.VMEM((_PAD_P, cols), jnp.float32),
            pltpu.VMEM((8, _LANES), jnp.float32),
            pltpu.VMEM((_ACT, cols), jnp.float32),
            pltpu.SemaphoreType.DMA,
            pltpu.SemaphoreType.DMA,
        ],
    )
    return sc_fn(simsT, wb).T


# speedup vs baseline: 1.0046x; 1.0046x over previous
import jax
import jax.numpy as jnp
from jax import lax
from jax.experimental import pallas as pl
from jax.experimental.pallas import tpu as pltpu
from jax.experimental.pallas import tpu_sc as plsc

_POINT = 11
_PAD_P = 16
_ACT = 4
_GROUPS = ((0, 4), (4, 9), (9, 10), (10, 11))

_BLK = 8192

_SC_INFO = plsc.get_sparse_core_info()
_NC = _SC_INFO.num_cores
_NW = _NC * _SC_INFO.num_subcores
_LANES = _SC_INFO.num_lanes


def _tc_body(x_ref, av_ref, w_ref, b_ref, out_ref, wb_ref):
    x = x_ref[...]
    av = av_ref[0]
    avn2 = jnp.sum(av * av, axis=0, keepdims=True)
    avs = av / jnp.maximum(jnp.sqrt(avn2), 1e-8)
    numT = lax.dot_general(avs, x, (((0,), (1,)), ((), ())),
                           preferred_element_type=jnp.float32)
    ones = jnp.ones((1, x_ref.shape[1]), jnp.float32)
    ssT = lax.dot_general(ones, x * x, (((1,), (1,)), ((), ())),
                          preferred_element_type=jnp.float32)
    simsT = numT / jnp.maximum(jnp.sqrt(ssT), 1e-8)
    out_ref[...] = jnp.concatenate(
        [simsT, jnp.zeros((_PAD_P - _POINT, x_ref.shape[0]), jnp.float32)], axis=0)

    @pl.when(pl.program_id(0) == 0)
    def _():
        w = w_ref[...]
        lane = lax.broadcasted_iota(jnp.int32, (1, _LANES), 1)
        row_b = jnp.zeros((1, _LANES), jnp.float32)
        for j in range(_ACT):
            row_b = jnp.where(lane == j, b_ref[j], row_b)
        pad_c = jnp.zeros((_ACT, _LANES - _ACT), jnp.float32)
        wb_ref[...] = jnp.concatenate(
            [jnp.concatenate([w, pad_c], axis=1),
             row_b,
             jnp.zeros((3, _LANES), jnp.float32)], axis=0)


def _sc_body(sims_hbm, wb_hbm, out_hbm, sims_v, wb_v, out_v, sem0, sem1):
    cols = sims_v.shape[1]
    half = cols // 2
    wid = lax.axis_index("s") * _NC + lax.axis_index("c")
    base = wid * cols
    cp0 = pltpu.async_copy(sims_hbm.at[:, pl.ds(base, half)],
                           sims_v.at[:, pl.ds(0, half)], sem0)
    cp1 = pltpu.async_copy(sims_hbm.at[:, pl.ds(base + half, half)],
                           sims_v.at[:, pl.ds(half, half)], sem1)
    pltpu.sync_copy(wb_hbm, wb_v)
    wrows = [wb_v[j] for j in range(_ACT + 1)]
    w = [[wrows[j][k] for k in range(_ACT)] for j in range(_ACT)]
    bias = [wrows[_ACT][j] for j in range(_ACT)]
    cp0.wait()
    for c in range(cols // _LANES):
        if c == (cols // _LANES) // 2:
            cp1.wait()
        sl = pl.ds(c * _LANES, _LANES)
        v = [sims_v[r, sl] for r in range(_POINT)]
        g = []
        for (s, e) in _GROUPS:
            m = v[s]
            for r in range(s + 1, e):
                m = jnp.maximum(m, v[r])
            g.append(m)
        logits = []
        for j in range(_ACT):
            l = bias[j]
            for k in range(_ACT):
                l = l + w[j][k] * g[k]
            logits.append(l)
        m = jnp.maximum(jnp.maximum(logits[0], logits[1]),
                        jnp.maximum(logits[2], logits[3]))
        exps = [jnp.exp(l - m) for l in logits]
        inv = 1.0 / (exps[0] + exps[1] + exps[2] + exps[3])
        for j in range(_ACT):
            out_v[j, sl] = exps[j] * inv
    pltpu.sync_copy(out_v, out_hbm.at[:, pl.ds(base, cols)])


def kernel(embedded_words, action_vectors, W, b):
    B, D = embedded_words.shape
    cols = B // _NW

    simsT, wb = pl.pallas_call(
        _tc_body,
        grid=(B // _BLK,),
        in_specs=[
            pl.BlockSpec((_BLK, D), lambda i: (i, 0)),
            pl.BlockSpec((1, D, _POINT), lambda i: (0, 0, 0)),
            pl.BlockSpec((_ACT, _ACT), lambda i: (0, 0)),
            pl.BlockSpec(memory_space=pltpu.SMEM),
        ],
        out_specs=[
            pl.BlockSpec((_PAD_P, _BLK), lambda i: (0, i)),
            pl.BlockSpec((8, _LANES), lambda i: (0, 0)),
        ],
        out_shape=[
            jax.ShapeDtypeStruct((_PAD_P, B), jnp.float32),
            jax.ShapeDtypeStruct((8, _LANES), jnp.float32),
        ],
    )(embedded_words, action_vectors, W, b)

    sc_fn = pl.kernel(
        _sc_body,
        out_type=jax.ShapeDtypeStruct((_ACT, B), jnp.float32),
        mesh=plsc.VectorSubcoreMesh(core_axis_name="c", subcore_axis_name="s"),
        scratch_types=[
            pltpu.VMEM((_PAD_P, cols), jnp.float32),
            pltpu.VMEM((8, _LANES), jnp.float32),
            pltpu.VMEM((_ACT, cols), jnp.float32),
            pltpu.SemaphoreType.DMA,
            pltpu.SemaphoreType.DMA,
        ],
    )
    return sc_fn(simsT, wb).T


# speedup vs baseline: 1.0055x; 1.0009x over previous
import jax
import jax.numpy as jnp
from jax import lax
from jax.experimental import pallas as pl
from jax.experimental.pallas import tpu as pltpu
from jax.experimental.pallas import tpu_sc as plsc

_POINT = 11
_PAD_P = 16
_ACT = 4
_GROUPS = ((0, 4), (4, 9), (9, 10), (10, 11))

_BLK = 8192

_SC_INFO = plsc.get_sparse_core_info()
_NC = _SC_INFO.num_cores
_NW = _NC * _SC_INFO.num_subcores
_LANES = _SC_INFO.num_lanes


def _tc_body(x_ref, av_ref, w_ref, b_ref, out_ref, wb_ref):
    x = x_ref[...]
    avT = av_ref[...]
    avn2 = jnp.sum(avT * avT, axis=1, keepdims=True)
    avsT = avT / jnp.maximum(jnp.sqrt(avn2), 1e-8)
    numT = lax.dot_general(avsT, x, (((1,), (1,)), ((), ())),
                           preferred_element_type=jnp.float32)
    ones = jnp.ones((1, x_ref.shape[1]), jnp.float32)
    ssT = lax.dot_general(ones, x * x, (((1,), (1,)), ((), ())),
                          preferred_element_type=jnp.float32)
    simsT = numT / jnp.maximum(jnp.sqrt(ssT), 1e-8)
    out_ref[...] = jnp.concatenate(
        [simsT, jnp.zeros((_PAD_P - _POINT, x_ref.shape[0]), jnp.float32)], axis=0)

    @pl.when(pl.program_id(0) == 0)
    def _():
        w = w_ref[...]
        lane = lax.broadcasted_iota(jnp.int32, (1, _LANES), 1)
        row_b = jnp.zeros((1, _LANES), jnp.float32)
        for j in range(_ACT):
            row_b = jnp.where(lane == j, b_ref[j], row_b)
        pad_c = jnp.zeros((_ACT, _LANES - _ACT), jnp.float32)
        wb_ref[...] = jnp.concatenate(
            [jnp.concatenate([w, pad_c], axis=1),
             row_b,
             jnp.zeros((3, _LANES), jnp.float32)], axis=0)


def _sc_body(sims_hbm, wb_hbm, out_hbm, sims_v, wb_v, out_v, sem0, sem1):
    cols = sims_v.shape[1]
    half = cols // 2
    wid = lax.axis_index("s") * _NC + lax.axis_index("c")
    base = wid * cols
    cp0 = pltpu.async_copy(sims_hbm.at[:, pl.ds(base, half)],
                           sims_v.at[:, pl.ds(0, half)], sem0)
    cp1 = pltpu.async_copy(sims_hbm.at[:, pl.ds(base + half, half)],
                           sims_v.at[:, pl.ds(half, half)], sem1)
    pltpu.sync_copy(wb_hbm, wb_v)
    wrows = [wb_v[j] for j in range(_ACT + 1)]
    w = [[wrows[j][k] for k in range(_ACT)] for j in range(_ACT)]
    bias = [wrows[_ACT][j] for j in range(_ACT)]
    cp0.wait()
    for c in range(cols // _LANES):
        if c == (cols // _LANES) // 2:
            cp1.wait()
        sl = pl.ds(c * _LANES, _LANES)
        v = [sims_v[r, sl] for r in range(_POINT)]
        g = []
        for (s, e) in _GROUPS:
            m = v[s]
            for r in range(s + 1, e):
                m = jnp.maximum(m, v[r])
            g.append(m)
        logits = []
        for j in range(_ACT):
            l = bias[j]
            for k in range(_ACT):
                l = l + w[j][k] * g[k]
            logits.append(l)
        m = jnp.maximum(jnp.maximum(logits[0], logits[1]),
                        jnp.maximum(logits[2], logits[3]))
        exps = [jnp.exp(l - m) for l in logits]
        inv = 1.0 / (exps[0] + exps[1] + exps[2] + exps[3])
        for j in range(_ACT):
            out_v[j, sl] = exps[j] * inv
    pltpu.sync_copy(out_v, out_hbm.at[:, pl.ds(base, cols)])


def kernel(embedded_words, action_vectors, W, b):
    B, D = embedded_words.shape
    cols = B // _NW

    simsT, wb = pl.pallas_call(
        _tc_body,
        grid=(B // _BLK,),
        in_specs=[
            pl.BlockSpec((_BLK, D), lambda i: (i, 0)),
            pl.BlockSpec((_POINT, D), lambda i: (0, 0)),
            pl.BlockSpec((_ACT, _ACT), lambda i: (0, 0)),
            pl.BlockSpec(memory_space=pltpu.SMEM),
        ],
        out_specs=[
            pl.BlockSpec((_PAD_P, _BLK), lambda i: (0, i)),
            pl.BlockSpec((8, _LANES), lambda i: (0, 0)),
        ],
        out_shape=[
            jax.ShapeDtypeStruct((_PAD_P, B), jnp.float32),
            jax.ShapeDtypeStruct((8, _LANES), jnp.float32),
        ],
    )(embedded_words, action_vectors[0].T, W, b)

    sc_fn = pl.kernel(
        _sc_body,
        out_type=jax.ShapeDtypeStruct((_ACT, B), jnp.float32),
        mesh=plsc.VectorSubcoreMesh(core_axis_name="c", subcore_axis_name="s"),
        scratch_types=[
            pltpu.VMEM((_PAD_P, cols), jnp.float32),
            pltpu.VMEM((8, _LANES), jnp.float32),
            pltpu.VMEM((_ACT, cols), jnp.float32),
            pltpu.SemaphoreType.DMA,
            pltpu.SemaphoreType.DMA,
        ],
    )
    return sc_fn(simsT, wb).T
